# Initial kernel scaffold; baseline (speedup 1.0000x reference)
import jax
import jax.numpy as jnp
from jax.experimental import pallas as pl


def kernel(x, edge_index, batch, W0, b0, p0, W1, b1, p1, W2, b2, p2, Wm, bm):
    raise NotImplementedError("write your pallas kernel here")


# speedup vs baseline: 25.0886x; 25.0886x over previous
import functools

import jax
import jax.numpy as jnp
from jax import lax
from jax.experimental import pallas as pl
from jax.experimental.pallas import tpu as pltpu
from jax.experimental.pallas import tpu_sc as plsc

N = 10000
E = 320000
F = 128
G = 16
NC, NS = 2, 16
NW = NC * NS
EPW = E // NW
CH = 80
NCHUNK = EPW // CH
NPAD = 10240
ROWS_PT = NPAD // NS
R = 1000
NBLK = N // R

@functools.cache
def _sc_mesh():
    return plsc.VectorSubcoreMesh(core_axis_name="c", subcore_axis_name="s",
                                  num_cores=NC, num_subcores=NS)


def _deg_body(src_hbm, dst_hbm, m_hbm, out_hbm, m_v, sidx, didx, vals, zv,
              deg_sh, sem):
    c = lax.axis_index("c")
    s = lax.axis_index("s")
    w = s * NC + c
    pltpu.sync_copy(m_hbm, m_v)
    for j in range(ROWS_PT // 16):
        zv[pl.ds(j * 16, 16)] = jnp.zeros((16,), jnp.float32)
    pltpu.sync_copy(zv, deg_sh.at[pl.ds(s * ROWS_PT, ROWS_PT)])
    plsc.subcore_barrier()

    base0 = w * EPW

    def body(i, carry):
        base = base0 + i * CH
        pltpu.sync_copy(src_hbm.at[pl.ds(base, CH)], sidx)
        pltpu.sync_copy(dst_hbm.at[pl.ds(base, CH)], didx)
        for j in range(CH // 16):
            idx = sidx[pl.ds(j * 16, 16)]
            vals[pl.ds(j * 16, 16)] = plsc.load_gather(m_v, [idx])
        pltpu.sync_copy(vals, deg_sh.at[didx], add=True)
        return carry

    lax.fori_loop(0, NCHUNK, body, 0)
    plsc.subcore_barrier()
    pltpu.sync_copy(deg_sh.at[pl.ds(s * ROWS_PT, ROWS_PT)],
                    out_hbm.at[c, pl.ds(s * ROWS_PT, ROWS_PT)])


@functools.cache
def _deg_kernel():
    return pl.kernel(
        _deg_body,
        out_type=jax.ShapeDtypeStruct((NC, NPAD), jnp.float32),
        mesh=_sc_mesh(),
        compiler_params=pltpu.CompilerParams(use_tc_tiling_on_sc=False, needs_layout_passes=False),
        scratch_types=[
            pltpu.VMEM((N,), jnp.float32),
            pltpu.VMEM((CH,), jnp.int32),
            pltpu.VMEM((CH,), jnp.int32),
            pltpu.VMEM((CH,), jnp.float32),
            pltpu.VMEM((ROWS_PT,), jnp.float32),
            pltpu.VMEM_SHARED((NPAD,), jnp.float32),
            pltpu.SemaphoreType.DMA,
        ],
    )


def _agg_body(src_hbm, dst_hbm, hp_hbm, out_hbm, sidx, didx, rows, zrow,
              agg_sh, sem):
    c = lax.axis_index("c")
    s = lax.axis_index("s")
    w = s * NC + c
    for i in range(16):
        for j in range(F // 16):
            zrow[i, pl.ds(j * 16, 16)] = jnp.zeros((16,), jnp.float32)

    def zloop(i, carry):
        pltpu.sync_copy(zrow, agg_sh.at[pl.ds(s * ROWS_PT + i * 16, 16), :])
        return carry

    lax.fori_loop(0, ROWS_PT // 16, zloop, 0)
    plsc.subcore_barrier()

    base0 = w * EPW

    def body(i, carry):
        base = base0 + i * CH
        pltpu.sync_copy(src_hbm.at[pl.ds(base, CH)], sidx)
        pltpu.sync_copy(dst_hbm.at[pl.ds(base, CH)], didx)
        pltpu.async_copy(hp_hbm.at[sidx], rows, sem).wait()
        pltpu.sync_copy(rows, agg_sh.at[didx], add=True)
        return carry

    lax.fori_loop(0, NCHUNK, body, 0)
    plsc.subcore_barrier()
    pltpu.sync_copy(agg_sh.at[pl.ds(s * ROWS_PT, ROWS_PT), :],
                    out_hbm.at[c, pl.ds(s * ROWS_PT, ROWS_PT), :])


@functools.cache
def _agg_kernel():
    return pl.kernel(
        _agg_body,
        out_type=jax.ShapeDtypeStruct((NC, NPAD, F), jnp.float32),
        mesh=_sc_mesh(),
        compiler_params=pltpu.CompilerParams(use_tc_tiling_on_sc=False, needs_layout_passes=False),
        scratch_types=[
            pltpu.VMEM((CH,), jnp.int32),
            pltpu.VMEM((CH,), jnp.int32),
            pltpu.VMEM((CH, F), jnp.float32),
            pltpu.VMEM((16, F), jnp.float32),
            pltpu.VMEM_SHARED((NPAD, F), jnp.float32),
            pltpu.SemaphoreType.DMA,
        ],
    )


def _hp_body(x_ref, w_ref, sf_ref, degp_ref, m_ref, hp_ref, dinv_ref):
    xb = x_ref[...] * sf_ref[...]
    h = jnp.dot(xb, w_ref[...], preferred_element_type=jnp.float32)
    deg = 1.0 + degp_ref[0] + degp_ref[1]
    dinv = lax.rsqrt(deg)
    hp_ref[...] = h * (dinv * m_ref[...])
    dinv_ref[...] = dinv


def _hp_call(x, w, sf, degp, m):
    return pl.pallas_call(
        _hp_body,
        grid=(NBLK,),
        in_specs=[
            pl.BlockSpec((R, F), lambda i: (i, 0)),
            pl.BlockSpec((F, F), lambda i: (0, 0)),
            pl.BlockSpec((R, 1), lambda i: (i, 0)),
            pl.BlockSpec((NC, R, 1), lambda i: (0, i, 0)),
            pl.BlockSpec((R, 1), lambda i: (i, 0)),
        ],
        out_specs=[
            pl.BlockSpec((R, F), lambda i: (i, 0)),
            pl.BlockSpec((R, 1), lambda i: (i, 0)),
        ],
        out_shape=[
            jax.ShapeDtypeStruct((N, F), jnp.float32),
            jax.ShapeDtypeStruct((N, 1), jnp.float32),
        ],
    )(x, w, sf, degp, m)


def _comb_body(aggp_ref, hp_ref, dinv_ref, m_ref, b_ref, p_ref, xc_ref,
               sv_ref):
    agg = aggp_ref[0] + aggp_ref[1]
    xc = (agg + hp_ref[...]) * dinv_ref[...] + b_ref[...]
    xc = jnp.maximum(xc, 0.0) * m_ref[...]
    xc_ref[...] = xc
    p = p_ref[...]
    pn = 1.0 / jnp.sqrt(jnp.sum(p * p))
    mv = lax.dot_general(xc, p, (((1,), (0,)), ((), ())),
                         preferred_element_type=jnp.float32)
    sv_ref[...] = jnp.tanh(mv * pn)


def _comb_call(aggp, hp, dinv, m, b, p):
    return pl.pallas_call(
        _comb_body,
        grid=(NBLK,),
        in_specs=[
            pl.BlockSpec((NC, R, F), lambda i: (0, i, 0)),
            pl.BlockSpec((R, F), lambda i: (i, 0)),
            pl.BlockSpec((R, 1), lambda i: (i, 0)),
            pl.BlockSpec((R, 1), lambda i: (i, 0)),
            pl.BlockSpec((F,), lambda i: (0,)),
            pl.BlockSpec((F, 1), lambda i: (0, 0)),
        ],
        out_specs=[
            pl.BlockSpec((R, F), lambda i: (i, 0)),
            pl.BlockSpec((R, 1), lambda i: (i, 0)),
        ],
        out_shape=[
            jax.ShapeDtypeStruct((N, F), jnp.float32),
            jax.ShapeDtypeStruct((N, 1), jnp.float32),
        ],
    )(aggp, hp, dinv, m, b, p)


def _topk_body(sv_ref, batch_ref, m_ref, sel_ref, sfac_ref):
    sv = sv_ref[...]
    valid = m_ref[...] > 0.0
    oh = batch_ref[...] == lax.broadcasted_iota(jnp.int32, (1, G), 1)
    validg = valid & oh
    counts = jnp.sum(validg.astype(jnp.float32), axis=0)
    k = jnp.floor((counts + 1.0) * 0.5)

    ub = lax.bitcast_convert_type(sv, jnp.uint32)
    neg = ub >= jnp.uint32(0x80000000)
    ukey = jnp.where(neg, ~ub, ub | jnp.uint32(0x80000000))
    ukey = jnp.where(valid, ukey, jnp.uint32(0))

    def step(t, tv):
        bit = jnp.uint32(1) << (jnp.uint32(31) - t.astype(jnp.uint32))
        cand = tv | bit
        ind = (ukey >= cand[None, :]) & validg
        cge = jnp.sum(ind.astype(jnp.float32), axis=0)
        return jnp.where(cge >= k, cand, tv)

    tv = lax.fori_loop(0, 32, step, jnp.zeros((G,), jnp.uint32))

    gtt = (ukey > tv[None, :]) & validg
    c1 = jnp.sum(gtt.astype(jnp.float32), axis=0)
    eq = (ukey == tv[None, :]) & validg
    need = k - c1
    idx = lax.broadcasted_iota(jnp.int32, (N, 1), 0)

    def step2(t, iv):
        bit = jnp.int32(1) << (13 - t)
        cand = iv + bit
        ind = eq & (idx < cand[None, :])
        cle = jnp.sum(ind.astype(jnp.float32), axis=0)
        return jnp.where(cle <= need, cand, iv)

    iv = lax.fori_loop(0, 14, step2, jnp.zeros((G,), jnp.int32))
    tie = eq & (idx < iv[None, :])
    sel = jnp.sum((gtt | tie).astype(jnp.float32), axis=1, keepdims=True)
    sel_ref[...] = sel
    sfac_ref[...] = sel * sv


def _topk_call(sv, batch, m):
    return pl.pallas_call(
        _topk_body,
        out_shape=[
            jax.ShapeDtypeStruct((N, 1), jnp.float32),
            jax.ShapeDtypeStruct((N, 1), jnp.float32),
        ],
    )(sv, batch, m)


def _final_body(xc_ref, sfac_ref, batch_ref, wm_ref, bm_ref, out_ref):
    xs = xc_ref[...] * sfac_ref[...]
    oh = (batch_ref[...] ==
          lax.broadcasted_iota(jnp.int32, (1, G), 1)).astype(jnp.float32)
    pooled = lax.dot_general(oh, xs, (((0,), (0,)), ((), ())),
                             preferred_element_type=jnp.float32)
    out_ref[...] = jnp.dot(pooled, wm_ref[...],
                           preferred_element_type=jnp.float32) + bm_ref[...]


def _final_call(xc, sfac, batch, wm, bm):
    return pl.pallas_call(
        _final_body,
        out_shape=jax.ShapeDtypeStruct((G, wm.shape[1]), jnp.float32),
    )(xc, sfac, batch, wm, bm)


def kernel(x, edge_index, batch, W0, b0, p0, W1, b1, p1, W2, b2, p2, Wm, bm):
    src = edge_index[0]
    dst = edge_index[1]
    bt2 = batch[:, None]
    m2 = jnp.ones((N, 1), jnp.float32)
    sf2 = jnp.ones((N, 1), jnp.float32)
    for (w, b, p) in ((W0, b0, p0), (W1, b1, p1), (W2, b2, p2)):
        degp = _deg_kernel()(src, dst, m2.reshape(N))
        hp, dinv = _hp_call(x, w, sf2, degp[:, :, None], m2)
        aggp = _agg_kernel()(src, dst, hp)
        xc, sv = _comb_call(aggp, hp, dinv, m2, b, p[:, None])
        sel, sf2 = _topk_call(sv, bt2, m2)
        x, m2 = xc, sel
    return _final_call(x, sf2, bt2, Wm, bm)


# speedup vs baseline: 26.0067x; 1.0366x over previous
import functools

import jax
import jax.numpy as jnp
from jax import lax
from jax.experimental import pallas as pl
from jax.experimental.pallas import tpu as pltpu
from jax.experimental.pallas import tpu_sc as plsc

N = 10000
E = 320000
F = 128
G = 16
NC, NS = 2, 16
NW = NC * NS
CH = 128
NCH = 80
EPT = NCH * CH
E2 = NW * EPT
NPAD = 10240
ROWS_PT = NPAD // NS
R = 1000
NBLK = N // R
IDX_NB = 4
AGG_NB = 2

@functools.cache
def _sc_mesh():
    return plsc.VectorSubcoreMesh(core_axis_name="c", subcore_axis_name="s",
                                  num_cores=NC, num_subcores=NS)


def _deg_body(ei_hbm, m_hbm, out_hbm, m_v, sd_all, vals, zv, deg_sh, sem):
    c = lax.axis_index("c")
    s = lax.axis_index("s")
    w = s * NC + c
    pltpu.sync_copy(m_hbm, m_v)
    pltpu.sync_copy(ei_hbm.at[w], sd_all)
    for j in range(ROWS_PT // 16):
        zv[pl.ds(j * 16, 16)] = jnp.zeros((16,), jnp.float32)
    pltpu.sync_copy(zv, deg_sh.at[pl.ds(s * ROWS_PT, ROWS_PT)])
    plsc.subcore_barrier()

    def body(i, carry):
        for j in range(CH // 16):
            idx = sd_all[i, 0, pl.ds(j * 16, 16)]
            vals[pl.ds(j * 16, 16)] = plsc.load_gather(m_v, [idx])
        pltpu.sync_copy(vals, deg_sh.at[sd_all.at[i].at[1]], add=True)
        return carry

    lax.fori_loop(0, NCH, body, 0)
    plsc.subcore_barrier()
    pltpu.sync_copy(deg_sh.at[pl.ds(s * ROWS_PT, ROWS_PT)],
                    out_hbm.at[c, pl.ds(s * ROWS_PT, ROWS_PT)])


@functools.cache
def _deg_kernel():
    return pl.kernel(
        _deg_body,
        out_type=jax.ShapeDtypeStruct((NC, NPAD), jnp.float32),
        mesh=_sc_mesh(),
        compiler_params=pltpu.CompilerParams(use_tc_tiling_on_sc=False, needs_layout_passes=False),
        scratch_types=[
            pltpu.VMEM((N,), jnp.float32),
            pltpu.VMEM((NCH, 2, CH), jnp.int32),
            pltpu.VMEM((CH,), jnp.float32),
            pltpu.VMEM((ROWS_PT,), jnp.float32),
            pltpu.VMEM_SHARED((NPAD,), jnp.float32),
            pltpu.SemaphoreType.DMA,
        ],
    )


def _agg_body(ei_hbm, hp_hbm, out_hbm, sd, rows, zrow, agg_sh, semi, semg):
    c = lax.axis_index("c")
    s = lax.axis_index("s")
    w = s * NC + c
    for i in range(16):
        for j in range(F // 16):
            zrow[i, pl.ds(j * 16, 16)] = jnp.zeros((16,), jnp.float32)

    def zloop(i, carry):
        pltpu.sync_copy(zrow, agg_sh.at[pl.ds(s * ROWS_PT + i * 16, 16), :])
        return carry

    lax.fori_loop(0, ROWS_PT // 16, zloop, 0)
    plsc.subcore_barrier()

    for b in range(IDX_NB):
        pltpu.async_copy(ei_hbm.at[w, b], sd.at[b], semi.at[b])
    for b in range(AGG_NB):
        pltpu.make_async_copy(ei_hbm.at[w, b], sd.at[b], semi.at[b]).wait()
        pltpu.async_copy(hp_hbm.at[sd.at[b].at[0]], rows.at[b], semg.at[b])

    def outer(j, carry):
        for b in range(IDX_NB):
            i = j * IDX_NB + b
            gb = b % AGG_NB
            b2 = (b + AGG_NB) % IDX_NB
            pltpu.make_async_copy(hp_hbm.at[sd.at[b].at[0]], rows.at[gb],
                                  semg.at[gb]).wait()
            pltpu.sync_copy(rows.at[gb], agg_sh.at[sd.at[b].at[1]], add=True)

            @pl.when(i + IDX_NB < NCH)
            def _():
                pltpu.async_copy(ei_hbm.at[w, i + IDX_NB], sd.at[b],
                                 semi.at[b])

            @pl.when(i + AGG_NB < NCH)
            def _():
                pltpu.make_async_copy(ei_hbm.at[w, i + AGG_NB], sd.at[b2],
                                      semi.at[b2]).wait()
                pltpu.async_copy(hp_hbm.at[sd.at[b2].at[0]], rows.at[gb],
                                 semg.at[gb])
        return carry

    lax.fori_loop(0, NCH // IDX_NB, outer, 0)
    plsc.subcore_barrier()
    pltpu.sync_copy(agg_sh.at[pl.ds(s * ROWS_PT, ROWS_PT), :],
                    out_hbm.at[c, pl.ds(s * ROWS_PT, ROWS_PT), :])


@functools.cache
def _agg_kernel():
    return pl.kernel(
        _agg_body,
        out_type=jax.ShapeDtypeStruct((NC, NPAD, F), jnp.float32),
        mesh=_sc_mesh(),
        compiler_params=pltpu.CompilerParams(use_tc_tiling_on_sc=False, needs_layout_passes=False),
        scratch_types=[
            pltpu.VMEM((IDX_NB, 2, CH), jnp.int32),
            pltpu.VMEM((AGG_NB, CH, F), jnp.float32),
            pltpu.VMEM((16, F), jnp.float32),
            pltpu.VMEM_SHARED((NPAD, F), jnp.float32),
            pltpu.SemaphoreType.DMA((IDX_NB,)),
            pltpu.SemaphoreType.DMA((AGG_NB,)),
        ],
    )


def _hp_body(x_ref, w_ref, sf_ref, degp_ref, m_ref, hp_ref, dinv_ref):
    xb = x_ref[...] * sf_ref[...]
    h = jnp.dot(xb, w_ref[...], preferred_element_type=jnp.float32)
    deg = 1.0 + degp_ref[0] + degp_ref[1]
    dinv = lax.rsqrt(deg)
    hp_ref[...] = h * (dinv * m_ref[...])
    dinv_ref[...] = dinv


def _hp_call(x, w, sf, degp, m):
    return pl.pallas_call(
        _hp_body,
        grid=(NBLK,),
        in_specs=[
            pl.BlockSpec((R, F), lambda i: (i, 0)),
            pl.BlockSpec((F, F), lambda i: (0, 0)),
            pl.BlockSpec((R, 1), lambda i: (i, 0)),
            pl.BlockSpec((NC, R, 1), lambda i: (0, i, 0)),
            pl.BlockSpec((R, 1), lambda i: (i, 0)),
        ],
        out_specs=[
            pl.BlockSpec((R, F), lambda i: (i, 0)),
            pl.BlockSpec((R, 1), lambda i: (i, 0)),
        ],
        out_shape=[
            jax.ShapeDtypeStruct((N, F), jnp.float32),
            jax.ShapeDtypeStruct((N, 1), jnp.float32),
        ],
    )(x, w, sf, degp, m)


def _comb_body(aggp_ref, hp_ref, dinv_ref, m_ref, b_ref, p_ref, xc_ref,
               sv_ref):
    agg = aggp_ref[0] + aggp_ref[1]
    xc = (agg + hp_ref[...]) * dinv_ref[...] + b_ref[...]
    xc = jnp.maximum(xc, 0.0) * m_ref[...]
    xc_ref[...] = xc
    p = p_ref[...]
    pn = 1.0 / jnp.sqrt(jnp.sum(p * p))
    mv = lax.dot_general(xc, p, (((1,), (0,)), ((), ())),
                         preferred_element_type=jnp.float32)
    sv_ref[...] = jnp.tanh(mv * pn)


def _comb_call(aggp, hp, dinv, m, b, p):
    return pl.pallas_call(
        _comb_body,
        grid=(NBLK,),
        in_specs=[
            pl.BlockSpec((NC, R, F), lambda i: (0, i, 0)),
            pl.BlockSpec((R, F), lambda i: (i, 0)),
            pl.BlockSpec((R, 1), lambda i: (i, 0)),
            pl.BlockSpec((R, 1), lambda i: (i, 0)),
            pl.BlockSpec((F,), lambda i: (0,)),
            pl.BlockSpec((F, 1), lambda i: (0, 0)),
        ],
        out_specs=[
            pl.BlockSpec((R, F), lambda i: (i, 0)),
            pl.BlockSpec((R, 1), lambda i: (i, 0)),
        ],
        out_shape=[
            jax.ShapeDtypeStruct((N, F), jnp.float32),
            jax.ShapeDtypeStruct((N, 1), jnp.float32),
        ],
    )(aggp, hp, dinv, m, b, p)


def _topk_body(sv_ref, batch_ref, m_ref, sel_ref, sfac_ref):
    sv = sv_ref[...]
    valid = m_ref[...] > 0.0
    oh = batch_ref[...] == lax.broadcasted_iota(jnp.int32, (1, G), 1)
    validg = valid & oh
    counts = jnp.sum(validg.astype(jnp.float32), axis=0)
    k = jnp.floor((counts + 1.0) * 0.5)

    ub = lax.bitcast_convert_type(sv, jnp.uint32)
    neg = ub >= jnp.uint32(0x80000000)
    ukey = jnp.where(neg, ~ub, ub | jnp.uint32(0x80000000))
    ukey = jnp.where(valid, ukey, jnp.uint32(0))

    def step(t, tv):
        bit = jnp.uint32(1) << (jnp.uint32(31) - t.astype(jnp.uint32))
        cand = tv | bit
        ind = (ukey >= cand[None, :]) & validg
        cge = jnp.sum(ind.astype(jnp.float32), axis=0)
        return jnp.where(cge >= k, cand, tv)

    tv = lax.fori_loop(0, 32, step, jnp.zeros((G,), jnp.uint32))

    gtt = (ukey > tv[None, :]) & validg
    c1 = jnp.sum(gtt.astype(jnp.float32), axis=0)
    eq = (ukey == tv[None, :]) & validg
    need = k - c1
    idx = lax.broadcasted_iota(jnp.int32, (N, 1), 0)

    def step2(t, iv):
        bit = jnp.int32(1) << (13 - t)
        cand = iv + bit
        ind = eq & (idx < cand[None, :])
        cle = jnp.sum(ind.astype(jnp.float32), axis=0)
        return jnp.where(cle <= need, cand, iv)

    iv = lax.fori_loop(0, 14, step2, jnp.zeros((G,), jnp.int32))
    tie = eq & (idx < iv[None, :])
    sel = jnp.sum((gtt | tie).astype(jnp.float32), axis=1, keepdims=True)
    sel_ref[...] = sel
    sfac_ref[...] = sel * sv


def _topk_call(sv, batch, m):
    return pl.pallas_call(
        _topk_body,
        out_shape=[
            jax.ShapeDtypeStruct((N, 1), jnp.float32),
            jax.ShapeDtypeStruct((N, 1), jnp.float32),
        ],
    )(sv, batch, m)


def _final_body(xc_ref, sfac_ref, batch_ref, wm_ref, bm_ref, out_ref):
    xs = xc_ref[...] * sfac_ref[...]
    oh = (batch_ref[...] ==
          lax.broadcasted_iota(jnp.int32, (1, G), 1)).astype(jnp.float32)
    pooled = lax.dot_general(oh, xs, (((0,), (0,)), ((), ())),
                             preferred_element_type=jnp.float32)
    out_ref[...] = jnp.dot(pooled, wm_ref[...],
                           preferred_element_type=jnp.float32) + bm_ref[...]


def _final_call(xc, sfac, batch, wm, bm):
    return pl.pallas_call(
        _final_body,
        out_shape=jax.ShapeDtypeStruct((G, wm.shape[1]), jnp.float32),
    )(xc, sfac, batch, wm, bm)


def kernel(x, edge_index, batch, W0, b0, p0, W1, b1, p1, W2, b2, p2, Wm, bm):
    srcp = jnp.concatenate([edge_index[0], jnp.zeros((E2 - E,), jnp.int32)])
    dstp = jnp.concatenate(
        [edge_index[1], jnp.full((E2 - E,), NPAD - 1, jnp.int32)])
    ei4 = jnp.stack([srcp, dstp]).reshape(2, NW, NCH, CH).transpose(1, 2, 0, 3)
    bt2 = batch[:, None]
    m2 = jnp.ones((N, 1), jnp.float32)
    sf2 = jnp.ones((N, 1), jnp.float32)
    for (w, b, p) in ((W0, b0, p0), (W1, b1, p1), (W2, b2, p2)):
        degp = _deg_kernel()(ei4, m2.reshape(N))
        hp, dinv = _hp_call(x, w, sf2, degp[:, :, None], m2)
        aggp = _agg_kernel()(ei4, hp)
        xc, sv = _comb_call(aggp, hp, dinv, m2, b, p[:, None])
        sel, sf2 = _topk_call(sv, bt2, m2)
        x, m2 = xc, sel
    return _final_call(x, sf2, bt2, Wm, bm)


# speedup vs baseline: 43.4762x; 1.6717x over previous
import functools

import jax
import jax.numpy as jnp
from jax import lax
from jax.experimental import pallas as pl
from jax.experimental.pallas import tpu as pltpu
from jax.experimental.pallas import tpu_sc as plsc

N = 10000
E = 320000
F = 128
G = 16
NC, NS = 2, 16
NW = NC * NS
CH = 128
NCH = 80
EPT = NCH * CH
E2 = NW * EPT
NPAD = 10240
ROWS_PT = NPAD // NS
R = 1000
NBLK = N // R
IDX_NB = 4
AGG_NB = 2
F2 = F // 2
NCH2 = E2 // (NS * CH)
NROWS_T = N // NS

@functools.cache
def _sc_mesh():
    return plsc.VectorSubcoreMesh(core_axis_name="c", subcore_axis_name="s",
                                  num_cores=NC, num_subcores=NS)


def _deg_body(ei_hbm, m_hbm, out_hbm, m_v, sd_all, vals, zv, deg_sh, sem):
    c = lax.axis_index("c")
    s = lax.axis_index("s")
    w = s * NC + c
    pltpu.sync_copy(m_hbm, m_v)
    pltpu.sync_copy(ei_hbm.at[w], sd_all)
    for j in range(ROWS_PT // 16):
        zv[pl.ds(j * 16, 16)] = jnp.zeros((16,), jnp.float32)
    pltpu.sync_copy(zv, deg_sh.at[pl.ds(s * ROWS_PT, ROWS_PT)])
    plsc.subcore_barrier()

    def body(i, carry):
        for j in range(CH // 16):
            idx = sd_all[i, 0, pl.ds(j * 16, 16)]
            vals[pl.ds(j * 16, 16)] = plsc.load_gather(m_v, [idx])
        pltpu.sync_copy(vals, deg_sh.at[sd_all.at[i].at[1]], add=True)
        return carry

    lax.fori_loop(0, NCH, body, 0)
    plsc.subcore_barrier()
    pltpu.sync_copy(deg_sh.at[pl.ds(s * ROWS_PT, ROWS_PT)],
                    out_hbm.at[c, pl.ds(s * ROWS_PT, ROWS_PT)])


@functools.cache
def _deg_kernel():
    return pl.kernel(
        _deg_body,
        out_type=jax.ShapeDtypeStruct((NC, NPAD), jnp.float32),
        mesh=_sc_mesh(),
        compiler_params=pltpu.CompilerParams(use_tc_tiling_on_sc=False, needs_layout_passes=False),
        scratch_types=[
            pltpu.VMEM((N,), jnp.float32),
            pltpu.VMEM((NCH, 2, CH), jnp.int32),
            pltpu.VMEM((CH,), jnp.float32),
            pltpu.VMEM((ROWS_PT,), jnp.float32),
            pltpu.VMEM_SHARED((NPAD,), jnp.float32),
            pltpu.SemaphoreType.DMA,
        ],
    )


def _agg_body(ei_hbm, hp_hbm, out_hbm, sd, rows, zrow, hp_sh, agg_sh, semi,
              semg):
    c = lax.axis_index("c")
    s = lax.axis_index("s")
    pltpu.sync_copy(hp_hbm.at[c, pl.ds(s * NROWS_T, NROWS_T), :],
                    hp_sh.at[pl.ds(s * NROWS_T, NROWS_T), :])
    for i in range(16):
        for j in range(F2 // 16):
            zrow[i, pl.ds(j * 16, 16)] = jnp.zeros((16,), jnp.float32)

    def zloop(i, carry):
        pltpu.sync_copy(zrow, agg_sh.at[pl.ds(s * ROWS_PT + i * 16, 16), :])
        return carry

    lax.fori_loop(0, ROWS_PT // 16, zloop, 0)
    plsc.subcore_barrier()

    for b in range(IDX_NB):
        pltpu.async_copy(ei_hbm.at[s, b], sd.at[b], semi.at[b])
    for b in range(AGG_NB):
        pltpu.make_async_copy(ei_hbm.at[s, b], sd.at[b], semi.at[b]).wait()
        pltpu.async_copy(hp_sh.at[sd.at[b].at[0]], rows.at[b], semg.at[b])

    def outer(j, carry):
        for b in range(IDX_NB):
            i = j * IDX_NB + b
            gb = b % AGG_NB
            b2 = (b + AGG_NB) % IDX_NB
            pltpu.make_async_copy(hp_sh.at[sd.at[b].at[0]], rows.at[gb],
                                  semg.at[gb]).wait()
            pltpu.sync_copy(rows.at[gb], agg_sh.at[sd.at[b].at[1]], add=True)

            @pl.when(i + IDX_NB < NCH2)
            def _():
                pltpu.async_copy(ei_hbm.at[s, i + IDX_NB], sd.at[b],
                                 semi.at[b])

            @pl.when(i + AGG_NB < NCH2)
            def _():
                pltpu.make_async_copy(ei_hbm.at[s, i + AGG_NB], sd.at[b2],
                                      semi.at[b2]).wait()
                pltpu.async_copy(hp_sh.at[sd.at[b2].at[0]], rows.at[gb],
                                 semg.at[gb])
        return carry

    lax.fori_loop(0, NCH2 // IDX_NB, outer, 0)
    plsc.subcore_barrier()
    pltpu.sync_copy(agg_sh.at[pl.ds(s * ROWS_PT, ROWS_PT), :],
                    out_hbm.at[c, pl.ds(s * ROWS_PT, ROWS_PT), :])


@functools.cache
def _agg_kernel():
    return pl.kernel(
        _agg_body,
        out_type=jax.ShapeDtypeStruct((NC, NPAD, F2), jnp.float32),
        mesh=_sc_mesh(),
        compiler_params=pltpu.CompilerParams(use_tc_tiling_on_sc=False, needs_layout_passes=False),
        scratch_types=[
            pltpu.VMEM((IDX_NB, 2, CH), jnp.int32),
            pltpu.VMEM((AGG_NB, CH, F2), jnp.float32),
            pltpu.VMEM((16, F2), jnp.float32),
            pltpu.VMEM_SHARED((N, F2), jnp.float32),
            pltpu.VMEM_SHARED((NPAD, F2), jnp.float32),
            pltpu.SemaphoreType.DMA((IDX_NB,)),
            pltpu.SemaphoreType.DMA((AGG_NB,)),
        ],
    )


def _hp_body(x_ref, w_ref, sf_ref, degp_ref, m_ref, hp_ref, dinv_ref):
    xb = x_ref[...] * sf_ref[...]
    h = jnp.dot(xb, w_ref[...], preferred_element_type=jnp.float32)
    deg = 1.0 + degp_ref[0] + degp_ref[1]
    dinv = lax.rsqrt(deg)
    hp = h * (dinv * m_ref[...])
    hp_ref[0] = hp[:, :F2]
    hp_ref[1] = hp[:, F2:]
    dinv_ref[...] = dinv


def _hp_call(x, w, sf, degp, m):
    return pl.pallas_call(
        _hp_body,
        grid=(NBLK,),
        in_specs=[
            pl.BlockSpec((R, F), lambda i: (i, 0)),
            pl.BlockSpec((F, F), lambda i: (0, 0)),
            pl.BlockSpec((R, 1), lambda i: (i, 0)),
            pl.BlockSpec((NC, R, 1), lambda i: (0, i, 0)),
            pl.BlockSpec((R, 1), lambda i: (i, 0)),
        ],
        out_specs=[
            pl.BlockSpec((NC, R, F2), lambda i: (0, i, 0)),
            pl.BlockSpec((R, 1), lambda i: (i, 0)),
        ],
        out_shape=[
            jax.ShapeDtypeStruct((NC, N, F2), jnp.float32),
            jax.ShapeDtypeStruct((N, 1), jnp.float32),
        ],
    )(x, w, sf, degp, m)


def _comb_body(aggp_ref, hp_ref, dinv_ref, m_ref, b_ref, p_ref, xc_ref,
               sv_ref):
    agg = jnp.concatenate([aggp_ref[0] + hp_ref[0],
                           aggp_ref[1] + hp_ref[1]], axis=1)
    xc = agg * dinv_ref[...] + b_ref[...]
    xc = jnp.maximum(xc, 0.0) * m_ref[...]
    xc_ref[...] = xc
    p = p_ref[...]
    pn = 1.0 / jnp.sqrt(jnp.sum(p * p))
    mv = lax.dot_general(xc, p, (((1,), (0,)), ((), ())),
                         preferred_element_type=jnp.float32)
    sv_ref[...] = jnp.tanh(mv * pn)


def _comb_call(aggp, hp, dinv, m, b, p):
    return pl.pallas_call(
        _comb_body,
        grid=(NBLK,),
        in_specs=[
            pl.BlockSpec((NC, R, F2), lambda i: (0, i, 0)),
            pl.BlockSpec((NC, R, F2), lambda i: (0, i, 0)),
            pl.BlockSpec((R, 1), lambda i: (i, 0)),
            pl.BlockSpec((R, 1), lambda i: (i, 0)),
            pl.BlockSpec((F,), lambda i: (0,)),
            pl.BlockSpec((F, 1), lambda i: (0, 0)),
        ],
        out_specs=[
            pl.BlockSpec((R, F), lambda i: (i, 0)),
            pl.BlockSpec((R, 1), lambda i: (i, 0)),
        ],
        out_shape=[
            jax.ShapeDtypeStruct((N, F), jnp.float32),
            jax.ShapeDtypeStruct((N, 1), jnp.float32),
        ],
    )(aggp, hp, dinv, m, b, p)


def _topk_body(sv_ref, batch_ref, m_ref, sel_ref, sfac_ref):
    sv = sv_ref[...]
    valid = m_ref[...] > 0.0
    oh = batch_ref[...] == lax.broadcasted_iota(jnp.int32, (1, G), 1)
    validg = valid & oh
    counts = jnp.sum(validg.astype(jnp.float32), axis=0)
    k = jnp.floor((counts + 1.0) * 0.5)

    ub = lax.bitcast_convert_type(sv, jnp.uint32)
    neg = ub >= jnp.uint32(0x80000000)
    ukey = jnp.where(neg, ~ub, ub | jnp.uint32(0x80000000))
    ukey = jnp.where(valid, ukey, jnp.uint32(0))

    def step(t, tv):
        bit = jnp.uint32(1) << (jnp.uint32(31) - t.astype(jnp.uint32))
        cand = tv | bit
        ind = (ukey >= cand[None, :]) & validg
        cge = jnp.sum(ind.astype(jnp.float32), axis=0)
        return jnp.where(cge >= k, cand, tv)

    tv = lax.fori_loop(0, 32, step, jnp.zeros((G,), jnp.uint32))

    gtt = (ukey > tv[None, :]) & validg
    c1 = jnp.sum(gtt.astype(jnp.float32), axis=0)
    eq = (ukey == tv[None, :]) & validg
    need = k - c1
    idx = lax.broadcasted_iota(jnp.int32, (N, 1), 0)

    def step2(t, iv):
        bit = jnp.int32(1) << (13 - t)
        cand = iv + bit
        ind = eq & (idx < cand[None, :])
        cle = jnp.sum(ind.astype(jnp.float32), axis=0)
        return jnp.where(cle <= need, cand, iv)

    iv = lax.fori_loop(0, 14, step2, jnp.zeros((G,), jnp.int32))
    tie = eq & (idx < iv[None, :])
    sel = jnp.sum((gtt | tie).astype(jnp.float32), axis=1, keepdims=True)
    sel_ref[...] = sel
    sfac_ref[...] = sel * sv


def _topk_call(sv, batch, m):
    return pl.pallas_call(
        _topk_body,
        out_shape=[
            jax.ShapeDtypeStruct((N, 1), jnp.float32),
            jax.ShapeDtypeStruct((N, 1), jnp.float32),
        ],
    )(sv, batch, m)


def _final_body(xc_ref, sfac_ref, batch_ref, wm_ref, bm_ref, out_ref):
    xs = xc_ref[...] * sfac_ref[...]
    oh = (batch_ref[...] ==
          lax.broadcasted_iota(jnp.int32, (1, G), 1)).astype(jnp.float32)
    pooled = lax.dot_general(oh, xs, (((0,), (0,)), ((), ())),
                             preferred_element_type=jnp.float32)
    out_ref[...] = jnp.dot(pooled, wm_ref[...],
                           preferred_element_type=jnp.float32) + bm_ref[...]


def _final_call(xc, sfac, batch, wm, bm):
    return pl.pallas_call(
        _final_body,
        out_shape=jax.ShapeDtypeStruct((G, wm.shape[1]), jnp.float32),
    )(xc, sfac, batch, wm, bm)


def kernel(x, edge_index, batch, W0, b0, p0, W1, b1, p1, W2, b2, p2, Wm, bm):
    srcp = jnp.concatenate([edge_index[0], jnp.zeros((E2 - E,), jnp.int32)])
    dstp = jnp.concatenate(
        [edge_index[1], jnp.full((E2 - E,), NPAD - 1, jnp.int32)])
    ei4 = jnp.stack([srcp, dstp]).reshape(2, NW, NCH, CH).transpose(1, 2, 0, 3)
    ei5 = jnp.stack([srcp, dstp]).reshape(2, NS, NCH2, CH).transpose(
        1, 2, 0, 3)
    bt2 = batch[:, None]
    m2 = jnp.ones((N, 1), jnp.float32)
    sf2 = jnp.ones((N, 1), jnp.float32)
    for (w, b, p) in ((W0, b0, p0), (W1, b1, p1), (W2, b2, p2)):
        degp = _deg_kernel()(ei4, m2.reshape(N))
        hp, dinv = _hp_call(x, w, sf2, degp[:, :, None], m2)
        aggp = _agg_kernel()(ei5, hp)
        xc, sv = _comb_call(aggp, hp, dinv, m2, b, p[:, None])
        sel, sf2 = _topk_call(sv, bt2, m2)
        x, m2 = xc, sel
    return _final_call(x, sf2, bt2, Wm, bm)


# speedup vs baseline: 44.4014x; 1.0213x over previous
import functools

import jax
import jax.numpy as jnp
from jax import lax
from jax.experimental import pallas as pl
from jax.experimental.pallas import tpu as pltpu
from jax.experimental.pallas import tpu_sc as plsc

N = 10000
E = 320000
F = 128
G = 16
NC, NS = 2, 16
NW = NC * NS
CH = 128
NCH = 80
EPT = NCH * CH
E2 = NW * EPT
NPAD = 10240
ROWS_PT = NPAD // NS
R = 1000
NBLK = N // R
IDX_NB = 4
AGG_NB = 2
F2 = F // 2
NCH2 = E2 // (NS * CH)
NROWS_T = N // NS

@functools.cache
def _sc_mesh():
    return plsc.VectorSubcoreMesh(core_axis_name="c", subcore_axis_name="s",
                                  num_cores=NC, num_subcores=NS)


def _deg_body(ei_hbm, m_hbm, out_hbm, m_v, sd_all, vals, zv, deg_sh, sem):
    c = lax.axis_index("c")
    s = lax.axis_index("s")
    w = s * NC + c
    pltpu.sync_copy(m_hbm, m_v)
    pltpu.sync_copy(ei_hbm.at[w], sd_all)
    for j in range(ROWS_PT // 16):
        zv[pl.ds(j * 16, 16)] = jnp.zeros((16,), jnp.float32)
    pltpu.sync_copy(zv, deg_sh.at[pl.ds(s * ROWS_PT, ROWS_PT)])
    plsc.subcore_barrier()

    def body(i, carry):
        for j in range(CH // 16):
            idx = sd_all[i, 0, pl.ds(j * 16, 16)]
            vals[pl.ds(j * 16, 16)] = plsc.load_gather(m_v, [idx])
        pltpu.sync_copy(vals, deg_sh.at[sd_all.at[i].at[1]], add=True)
        return carry

    lax.fori_loop(0, NCH, body, 0)
    plsc.subcore_barrier()
    pltpu.sync_copy(deg_sh.at[pl.ds(s * ROWS_PT, ROWS_PT)],
                    out_hbm.at[c, pl.ds(s * ROWS_PT, ROWS_PT)])


@functools.cache
def _deg_kernel():
    return pl.kernel(
        _deg_body,
        out_type=jax.ShapeDtypeStruct((NC, NPAD), jnp.float32),
        mesh=_sc_mesh(),
        compiler_params=pltpu.CompilerParams(use_tc_tiling_on_sc=False, needs_layout_passes=False),
        scratch_types=[
            pltpu.VMEM((N,), jnp.float32),
            pltpu.VMEM((NCH, 2, CH), jnp.int32),
            pltpu.VMEM((CH,), jnp.float32),
            pltpu.VMEM((ROWS_PT,), jnp.float32),
            pltpu.VMEM_SHARED((NPAD,), jnp.float32),
            pltpu.SemaphoreType.DMA,
        ],
    )


def _agg_body(ei_hbm, hp_hbm, out_hbm, sd, rows, zrow, hp_sh, agg_sh, semi,
              semg):
    c = lax.axis_index("c")
    s = lax.axis_index("s")
    pltpu.sync_copy(hp_hbm.at[c, pl.ds(s * NROWS_T, NROWS_T), :],
                    hp_sh.at[pl.ds(s * NROWS_T, NROWS_T), :])
    for i in range(16):
        for j in range(F2 // 16):
            zrow[i, pl.ds(j * 16, 16)] = jnp.zeros((16,), jnp.float32)

    def zloop(i, carry):
        pltpu.sync_copy(zrow, agg_sh.at[pl.ds(s * ROWS_PT + i * 16, 16), :])
        return carry

    lax.fori_loop(0, ROWS_PT // 16, zloop, 0)
    plsc.subcore_barrier()

    for b in range(IDX_NB):
        pltpu.async_copy(ei_hbm.at[s, b], sd.at[b], semi.at[b])
    for b in range(AGG_NB):
        pltpu.make_async_copy(ei_hbm.at[s, b], sd.at[b], semi.at[b]).wait()
        pltpu.async_copy(hp_sh.at[sd.at[b].at[0]], rows.at[b], semg.at[b])

    def outer(j, carry):
        for b in range(IDX_NB):
            i = j * IDX_NB + b
            gb = b % AGG_NB
            b2 = (b + AGG_NB) % IDX_NB
            pltpu.make_async_copy(hp_sh.at[sd.at[b].at[0]], rows.at[gb],
                                  semg.at[gb]).wait()
            pltpu.sync_copy(rows.at[gb], agg_sh.at[sd.at[b].at[1]], add=True)

            @pl.when(i + IDX_NB < NCH2)
            def _():
                pltpu.async_copy(ei_hbm.at[s, i + IDX_NB], sd.at[b],
                                 semi.at[b])

            @pl.when(i + AGG_NB < NCH2)
            def _():
                pltpu.make_async_copy(ei_hbm.at[s, i + AGG_NB], sd.at[b2],
                                      semi.at[b2]).wait()
                pltpu.async_copy(hp_sh.at[sd.at[b2].at[0]], rows.at[gb],
                                 semg.at[gb])
        return carry

    lax.fori_loop(0, NCH2 // IDX_NB, outer, 0)
    plsc.subcore_barrier()
    pltpu.sync_copy(agg_sh.at[pl.ds(s * ROWS_PT, ROWS_PT), :],
                    out_hbm.at[c, pl.ds(s * ROWS_PT, ROWS_PT), :])


@functools.cache
def _agg_kernel():
    return pl.kernel(
        _agg_body,
        out_type=jax.ShapeDtypeStruct((NC, NPAD, F2), jnp.float32),
        mesh=_sc_mesh(),
        compiler_params=pltpu.CompilerParams(use_tc_tiling_on_sc=False, needs_layout_passes=False),
        scratch_types=[
            pltpu.VMEM((IDX_NB, 2, CH), jnp.int32),
            pltpu.VMEM((AGG_NB, CH, F2), jnp.float32),
            pltpu.VMEM((16, F2), jnp.float32),
            pltpu.VMEM_SHARED((N, F2), jnp.float32),
            pltpu.VMEM_SHARED((NPAD, F2), jnp.float32),
            pltpu.SemaphoreType.DMA((IDX_NB,)),
            pltpu.SemaphoreType.DMA((AGG_NB,)),
        ],
    )


def _hp_body(x_ref, w_ref, sf_ref, degp_ref, m_ref, hp_ref, dinv_ref):
    xb = x_ref[...] * sf_ref[...]
    h = jnp.dot(xb, w_ref[...], preferred_element_type=jnp.float32)
    deg = 1.0 + degp_ref[0] + degp_ref[1]
    dinv = lax.rsqrt(deg)
    hp = h * (dinv * m_ref[...])
    hp_ref[0] = hp[:, :F2]
    hp_ref[1] = hp[:, F2:]
    dinv_ref[...] = dinv


def _hp_call(x, w, sf, degp, m):
    return pl.pallas_call(
        _hp_body,
        grid=(NBLK,),
        in_specs=[
            pl.BlockSpec((R, F), lambda i: (i, 0)),
            pl.BlockSpec((F, F), lambda i: (0, 0)),
            pl.BlockSpec((R, 1), lambda i: (i, 0)),
            pl.BlockSpec((NC, R, 1), lambda i: (0, i, 0)),
            pl.BlockSpec((R, 1), lambda i: (i, 0)),
        ],
        out_specs=[
            pl.BlockSpec((NC, R, F2), lambda i: (0, i, 0)),
            pl.BlockSpec((R, 1), lambda i: (i, 0)),
        ],
        out_shape=[
            jax.ShapeDtypeStruct((NC, N, F2), jnp.float32),
            jax.ShapeDtypeStruct((N, 1), jnp.float32),
        ],
    )(x, w, sf, degp, m)


def _comb_body(aggp_ref, hp_ref, dinv_ref, m_ref, b_ref, p_ref, xc_ref,
               sv_ref):
    agg = jnp.concatenate([aggp_ref[0] + hp_ref[0],
                           aggp_ref[1] + hp_ref[1]], axis=1)
    xc = agg * dinv_ref[...] + b_ref[...]
    xc = jnp.maximum(xc, 0.0) * m_ref[...]
    xc_ref[...] = xc
    p = p_ref[...]
    pn = 1.0 / jnp.sqrt(jnp.sum(p * p))
    mv = lax.dot_general(xc, p, (((1,), (0,)), ((), ())),
                         preferred_element_type=jnp.float32)
    sv_ref[...] = jnp.tanh(mv * pn)


def _comb_call(aggp, hp, dinv, m, b, p):
    return pl.pallas_call(
        _comb_body,
        grid=(NBLK,),
        in_specs=[
            pl.BlockSpec((NC, R, F2), lambda i: (0, i, 0)),
            pl.BlockSpec((NC, R, F2), lambda i: (0, i, 0)),
            pl.BlockSpec((R, 1), lambda i: (i, 0)),
            pl.BlockSpec((R, 1), lambda i: (i, 0)),
            pl.BlockSpec((F,), lambda i: (0,)),
            pl.BlockSpec((F, 1), lambda i: (0, 0)),
        ],
        out_specs=[
            pl.BlockSpec((R, F), lambda i: (i, 0)),
            pl.BlockSpec((R, 1), lambda i: (i, 0)),
        ],
        out_shape=[
            jax.ShapeDtypeStruct((N, F), jnp.float32),
            jax.ShapeDtypeStruct((N, 1), jnp.float32),
        ],
    )(aggp, hp, dinv, m, b, p)


def _topk_body(sv_ref, batch_ref, m_ref, sel_ref, sfac_ref):
    sv = sv_ref[...]
    valid = m_ref[...] > 0.0
    oh = batch_ref[...] == lax.broadcasted_iota(jnp.int32, (1, G), 1)
    validg = (valid & oh).astype(jnp.float32)
    ones_row = jnp.ones((1, N), jnp.float32)

    def colsum(x):
        return lax.dot_general(ones_row, x, (((1,), (0,)), ((), ())),
                               preferred_element_type=jnp.float32)[0]

    counts = colsum(validg)
    k = jnp.floor((counts + 1.0) * 0.5)

    ub = lax.bitcast_convert_type(sv, jnp.uint32)
    neg = ub >= jnp.uint32(0x80000000)
    ukey = jnp.where(neg, ~ub, ub | jnp.uint32(0x80000000))
    ukey = jnp.where(valid, ukey, jnp.uint32(0))

    def step(t, tv):
        bit = jnp.uint32(1) << (jnp.uint32(31) - t.astype(jnp.uint32))
        cand = tv | bit
        cge = colsum((ukey >= cand[None, :]).astype(jnp.float32) * validg)
        return jnp.where(cge >= k, cand, tv)

    tv = lax.fori_loop(0, 32, step, jnp.zeros((G,), jnp.uint32))

    gtt = (ukey > tv[None, :]).astype(jnp.float32) * validg
    c1 = colsum(gtt)
    eq = (ukey == tv[None, :]).astype(jnp.float32) * validg
    need = k - c1
    idx = lax.broadcasted_iota(jnp.int32, (N, 1), 0)

    def step2(t, iv):
        bit = jnp.int32(1) << (13 - t)
        cand = iv + bit
        cle = colsum((idx < cand[None, :]).astype(jnp.float32) * eq)
        return jnp.where(cle <= need, cand, iv)

    iv = lax.fori_loop(0, 14, step2, jnp.zeros((G,), jnp.int32))
    tie = eq * (idx < iv[None, :]).astype(jnp.float32)
    sel = jnp.sum(jnp.minimum(gtt + tie, 1.0), axis=1, keepdims=True)
    sel_ref[...] = sel
    sfac_ref[...] = sel * sv


def _topk_call(sv, batch, m):
    return pl.pallas_call(
        _topk_body,
        out_shape=[
            jax.ShapeDtypeStruct((N, 1), jnp.float32),
            jax.ShapeDtypeStruct((N, 1), jnp.float32),
        ],
    )(sv, batch, m)


def _final_body(xc_ref, sfac_ref, batch_ref, wm_ref, bm_ref, out_ref):
    xs = xc_ref[...] * sfac_ref[...]
    oh = (batch_ref[...] ==
          lax.broadcasted_iota(jnp.int32, (1, G), 1)).astype(jnp.float32)
    pooled = lax.dot_general(oh, xs, (((0,), (0,)), ((), ())),
                             preferred_element_type=jnp.float32)
    out_ref[...] = jnp.dot(pooled, wm_ref[...],
                           preferred_element_type=jnp.float32) + bm_ref[...]


def _final_call(xc, sfac, batch, wm, bm):
    return pl.pallas_call(
        _final_body,
        out_shape=jax.ShapeDtypeStruct((G, wm.shape[1]), jnp.float32),
    )(xc, sfac, batch, wm, bm)


def kernel(x, edge_index, batch, W0, b0, p0, W1, b1, p1, W2, b2, p2, Wm, bm):
    srcp = jnp.concatenate([edge_index[0], jnp.zeros((E2 - E,), jnp.int32)])
    dstp = jnp.concatenate(
        [edge_index[1], jnp.full((E2 - E,), NPAD - 1, jnp.int32)])
    ei4 = jnp.stack([srcp, dstp]).reshape(2, NW, NCH, CH).transpose(1, 2, 0, 3)
    ei5 = jnp.stack([srcp, dstp]).reshape(2, NS, NCH2, CH).transpose(
        1, 2, 0, 3)
    bt2 = batch[:, None]
    m2 = jnp.ones((N, 1), jnp.float32)
    sf2 = jnp.ones((N, 1), jnp.float32)
    for (w, b, p) in ((W0, b0, p0), (W1, b1, p1), (W2, b2, p2)):
        degp = _deg_kernel()(ei4, m2.reshape(N))
        hp, dinv = _hp_call(x, w, sf2, degp[:, :, None], m2)
        aggp = _agg_kernel()(ei5, hp)
        xc, sv = _comb_call(aggp, hp, dinv, m2, b, p[:, None])
        sel, sf2 = _topk_call(sv, bt2, m2)
        x, m2 = xc, sel
    return _final_call(x, sf2, bt2, Wm, bm)


# speedup vs baseline: 48.3372x; 1.0886x over previous
import functools

import jax
import jax.numpy as jnp
from jax import lax
from jax.experimental import pallas as pl
from jax.experimental.pallas import tpu as pltpu
from jax.experimental.pallas import tpu_sc as plsc

N = 10000
E = 320000
F = 128
G = 16
NC, NS = 2, 16
NW = NC * NS
CH = 128
NCH = 80
EPT = NCH * CH
E2 = NW * EPT
NPAD = 10240
ROWS_PT = NPAD // NS
R = 1000
NBLK = N // R
IDX_NB = 8
AGG_NB = 4
F2 = F // 2
NCH2 = E2 // (NS * CH)
NROWS_T = N // NS

@functools.cache
def _sc_mesh():
    return plsc.VectorSubcoreMesh(core_axis_name="c", subcore_axis_name="s",
                                  num_cores=NC, num_subcores=NS)


def _deg_body(ei_hbm, m_hbm, out_hbm, m_v, sd_all, vals, zv, deg_sh, sem):
    c = lax.axis_index("c")
    s = lax.axis_index("s")
    w = s * NC + c
    pltpu.sync_copy(m_hbm, m_v)
    pltpu.sync_copy(ei_hbm.at[w], sd_all)
    for j in range(ROWS_PT // 16):
        zv[pl.ds(j * 16, 16)] = jnp.zeros((16,), jnp.float32)
    pltpu.sync_copy(zv, deg_sh.at[pl.ds(s * ROWS_PT, ROWS_PT)])
    plsc.subcore_barrier()

    def body(i, carry):
        for j in range(CH // 16):
            idx = sd_all[i, 0, pl.ds(j * 16, 16)]
            vals[pl.ds(j * 16, 16)] = plsc.load_gather(m_v, [idx])
        pltpu.sync_copy(vals, deg_sh.at[sd_all.at[i].at[1]], add=True)
        return carry

    lax.fori_loop(0, NCH, body, 0)
    plsc.subcore_barrier()
    pltpu.sync_copy(deg_sh.at[pl.ds(s * ROWS_PT, ROWS_PT)],
                    out_hbm.at[c, pl.ds(s * ROWS_PT, ROWS_PT)])


@functools.cache
def _deg_kernel():
    return pl.kernel(
        _deg_body,
        out_type=jax.ShapeDtypeStruct((NC, NPAD), jnp.float32),
        mesh=_sc_mesh(),
        compiler_params=pltpu.CompilerParams(use_tc_tiling_on_sc=False, needs_layout_passes=False),
        scratch_types=[
            pltpu.VMEM((N,), jnp.float32),
            pltpu.VMEM((NCH, 2, CH), jnp.int32),
            pltpu.VMEM((CH,), jnp.float32),
            pltpu.VMEM((ROWS_PT,), jnp.float32),
            pltpu.VMEM_SHARED((NPAD,), jnp.float32),
            pltpu.SemaphoreType.DMA,
        ],
    )


def _agg_body(ei_hbm, hp_hbm, out_hbm, sd, rows, zrow, hp_sh, agg_sh, semi,
              semg, sems):
    c = lax.axis_index("c")
    s = lax.axis_index("s")
    pltpu.sync_copy(hp_hbm.at[c, pl.ds(s * NROWS_T, NROWS_T), :],
                    hp_sh.at[pl.ds(s * NROWS_T, NROWS_T), :])
    for i in range(16):
        for j in range(F2 // 16):
            zrow[i, pl.ds(j * 16, 16)] = jnp.zeros((16,), jnp.float32)

    def zloop(i, carry):
        pltpu.sync_copy(zrow, agg_sh.at[pl.ds(s * ROWS_PT + i * 16, 16), :])
        return carry

    lax.fori_loop(0, ROWS_PT // 16, zloop, 0)
    plsc.subcore_barrier()

    for b in range(6):
        pltpu.async_copy(ei_hbm.at[s, b], sd.at[b], semi.at[b])
    for b in range(AGG_NB - 2):
        pltpu.make_async_copy(ei_hbm.at[s, b], sd.at[b], semi.at[b]).wait()
        pltpu.async_copy(hp_sh.at[sd.at[b].at[0]], rows.at[b], semg.at[b])

    def outer(j, carry):
        for b in range(IDX_NB):
            i = j * IDX_NB + b
            r = b % AGG_NB
            b2 = (b + 2) % IDX_NB
            r2 = (b + 2) % AGG_NB
            pltpu.make_async_copy(hp_sh.at[sd.at[b].at[0]], rows.at[r],
                                  semg.at[r]).wait()
            pltpu.async_copy(rows.at[r], agg_sh.at[sd.at[b].at[1]],
                             sems.at[r], add=True)

            @pl.when(i >= 2)
            def _():
                pltpu.make_async_copy(
                    rows.at[r2], agg_sh.at[sd.at[(b + 6) % IDX_NB].at[1]],
                    sems.at[r2]).wait()

            @pl.when(i + 2 < NCH2)
            def _():
                pltpu.make_async_copy(ei_hbm.at[s, i + 2], sd.at[b2],
                                      semi.at[b2]).wait()
                pltpu.async_copy(hp_sh.at[sd.at[b2].at[0]], rows.at[r2],
                                 semg.at[r2])

            @pl.when(i + 6 < NCH2)
            def _():
                pltpu.async_copy(ei_hbm.at[s, i + 6], sd.at[(b + 6) % IDX_NB],
                                 semi.at[(b + 6) % IDX_NB])
        return carry

    lax.fori_loop(0, NCH2 // IDX_NB, outer, 0)
    for b in range(2):
        bb = (NCH2 - 2 + b) % IDX_NB
        rr = (NCH2 - 2 + b) % AGG_NB
        pltpu.make_async_copy(rows.at[rr], agg_sh.at[sd.at[bb].at[1]],
                              sems.at[rr]).wait()
    plsc.subcore_barrier()
    pltpu.sync_copy(agg_sh.at[pl.ds(s * ROWS_PT, ROWS_PT), :],
                    out_hbm.at[c, pl.ds(s * ROWS_PT, ROWS_PT), :])


@functools.cache
def _agg_kernel():
    return pl.kernel(
        _agg_body,
        out_type=jax.ShapeDtypeStruct((NC, NPAD, F2), jnp.float32),
        mesh=_sc_mesh(),
        compiler_params=pltpu.CompilerParams(use_tc_tiling_on_sc=False, needs_layout_passes=False),
        scratch_types=[
            pltpu.VMEM((IDX_NB, 2, CH), jnp.int32),
            pltpu.VMEM((AGG_NB, CH, F2), jnp.float32),
            pltpu.VMEM((16, F2), jnp.float32),
            pltpu.VMEM_SHARED((N, F2), jnp.float32),
            pltpu.VMEM_SHARED((NPAD, F2), jnp.float32),
            pltpu.SemaphoreType.DMA((IDX_NB,)),
            pltpu.SemaphoreType.DMA((AGG_NB,)),
            pltpu.SemaphoreType.DMA((AGG_NB,)),
        ],
    )


def _hp_body(x_ref, w_ref, sf_ref, degp_ref, m_ref, hp_ref, dinv_ref):
    xb = x_ref[...] * sf_ref[...]
    h = jnp.dot(xb, w_ref[...], preferred_element_type=jnp.float32)
    deg = 1.0 + degp_ref[0] + degp_ref[1]
    dinv = lax.rsqrt(deg)
    hp = h * (dinv * m_ref[...])
    hp_ref[0] = hp[:, :F2]
    hp_ref[1] = hp[:, F2:]
    dinv_ref[...] = dinv


def _hp_call(x, w, sf, degp, m):
    return pl.pallas_call(
        _hp_body,
        grid=(NBLK,),
        in_specs=[
            pl.BlockSpec((R, F), lambda i: (i, 0)),
            pl.BlockSpec((F, F), lambda i: (0, 0)),
            pl.BlockSpec((R, 1), lambda i: (i, 0)),
            pl.BlockSpec((NC, R, 1), lambda i: (0, i, 0)),
            pl.BlockSpec((R, 1), lambda i: (i, 0)),
        ],
        out_specs=[
            pl.BlockSpec((NC, R, F2), lambda i: (0, i, 0)),
            pl.BlockSpec((R, 1), lambda i: (i, 0)),
        ],
        out_shape=[
            jax.ShapeDtypeStruct((NC, N, F2), jnp.float32),
            jax.ShapeDtypeStruct((N, 1), jnp.float32),
        ],
    )(x, w, sf, degp, m)


def _comb_body(aggp_ref, hp_ref, dinv_ref, m_ref, b_ref, p_ref, xc_ref,
               sv_ref):
    agg = jnp.concatenate([aggp_ref[0] + hp_ref[0],
                           aggp_ref[1] + hp_ref[1]], axis=1)
    xc = agg * dinv_ref[...] + b_ref[...]
    xc = jnp.maximum(xc, 0.0) * m_ref[...]
    xc_ref[...] = xc
    p = p_ref[...]
    pn = 1.0 / jnp.sqrt(jnp.sum(p * p))
    mv = lax.dot_general(xc, p, (((1,), (0,)), ((), ())),
                         preferred_element_type=jnp.float32)
    sv_ref[...] = jnp.tanh(mv * pn)


def _comb_call(aggp, hp, dinv, m, b, p):
    return pl.pallas_call(
        _comb_body,
        grid=(NBLK,),
        in_specs=[
            pl.BlockSpec((NC, R, F2), lambda i: (0, i, 0)),
            pl.BlockSpec((NC, R, F2), lambda i: (0, i, 0)),
            pl.BlockSpec((R, 1), lambda i: (i, 0)),
            pl.BlockSpec((R, 1), lambda i: (i, 0)),
            pl.BlockSpec((F,), lambda i: (0,)),
            pl.BlockSpec((F, 1), lambda i: (0, 0)),
        ],
        out_specs=[
            pl.BlockSpec((R, F), lambda i: (i, 0)),
            pl.BlockSpec((R, 1), lambda i: (i, 0)),
        ],
        out_shape=[
            jax.ShapeDtypeStruct((N, F), jnp.float32),
            jax.ShapeDtypeStruct((N, 1), jnp.float32),
        ],
    )(aggp, hp, dinv, m, b, p)


def _topk_body(sv_ref, batch_ref, m_ref, sel_ref, sfac_ref):
    sv = sv_ref[...]
    valid = m_ref[...] > 0.0
    oh = batch_ref[...] == lax.broadcasted_iota(jnp.int32, (1, G), 1)
    validg = (valid & oh).astype(jnp.float32)
    ones_row = jnp.ones((1, N), jnp.float32)

    def colsum(x):
        return lax.dot_general(ones_row, x, (((1,), (0,)), ((), ())),
                               preferred_element_type=jnp.float32)[0]

    counts = colsum(validg)
    k = jnp.floor((counts + 1.0) * 0.5)

    ub = lax.bitcast_convert_type(sv, jnp.uint32)
    neg = ub >= jnp.uint32(0x80000000)
    ukey = jnp.where(neg, ~ub, ub | jnp.uint32(0x80000000))
    ukey = jnp.where(valid, ukey, jnp.uint32(0))

    def step(t, tv):
        bit = jnp.uint32(1) << (jnp.uint32(31) - t.astype(jnp.uint32))
        cand = tv | bit
        cge = colsum((ukey >= cand[None, :]).astype(jnp.float32) * validg)
        return jnp.where(cge >= k, cand, tv)

    tv = lax.fori_loop(0, 32, step, jnp.zeros((G,), jnp.uint32))

    gtt = (ukey > tv[None, :]).astype(jnp.float32) * validg
    c1 = colsum(gtt)
    eq = (ukey == tv[None, :]).astype(jnp.float32) * validg
    need = k - c1
    idx = lax.broadcasted_iota(jnp.int32, (N, 1), 0)

    def step2(t, iv):
        bit = jnp.int32(1) << (13 - t)
        cand = iv + bit
        cle = colsum((idx < cand[None, :]).astype(jnp.float32) * eq)
        return jnp.where(cle <= need, cand, iv)

    iv = lax.fori_loop(0, 14, step2, jnp.zeros((G,), jnp.int32))
    tie = eq * (idx < iv[None, :]).astype(jnp.float32)
    sel = jnp.sum(jnp.minimum(gtt + tie, 1.0), axis=1, keepdims=True)
    sel_ref[...] = sel
    sfac_ref[...] = sel * sv


def _topk_call(sv, batch, m):
    return pl.pallas_call(
        _topk_body,
        out_shape=[
            jax.ShapeDtypeStruct((N, 1), jnp.float32),
            jax.ShapeDtypeStruct((N, 1), jnp.float32),
        ],
    )(sv, batch, m)


def _final_body(xc_ref, sfac_ref, batch_ref, wm_ref, bm_ref, out_ref):
    xs = xc_ref[...] * sfac_ref[...]
    oh = (batch_ref[...] ==
          lax.broadcasted_iota(jnp.int32, (1, G), 1)).astype(jnp.float32)
    pooled = lax.dot_general(oh, xs, (((0,), (0,)), ((), ())),
                             preferred_element_type=jnp.float32)
    out_ref[...] = jnp.dot(pooled, wm_ref[...],
                           preferred_element_type=jnp.float32) + bm_ref[...]


def _final_call(xc, sfac, batch, wm, bm):
    return pl.pallas_call(
        _final_body,
        out_shape=jax.ShapeDtypeStruct((G, wm.shape[1]), jnp.float32),
    )(xc, sfac, batch, wm, bm)


def kernel(x, edge_index, batch, W0, b0, p0, W1, b1, p1, W2, b2, p2, Wm, bm):
    srcp = jnp.concatenate([edge_index[0], jnp.zeros((E2 - E,), jnp.int32)])
    dstp = jnp.concatenate(
        [edge_index[1], jnp.full((E2 - E,), NPAD - 1, jnp.int32)])
    ei4 = jnp.stack([srcp, dstp]).reshape(2, NW, NCH, CH).transpose(1, 2, 0, 3)
    ei5 = jnp.stack([srcp, dstp]).reshape(2, NS, NCH2, CH).transpose(
        1, 2, 0, 3)
    bt2 = batch[:, None]
    m2 = jnp.ones((N, 1), jnp.float32)
    sf2 = jnp.ones((N, 1), jnp.float32)
    for (w, b, p) in ((W0, b0, p0), (W1, b1, p1), (W2, b2, p2)):
        degp = _deg_kernel()(ei4, m2.reshape(N))
        hp, dinv = _hp_call(x, w, sf2, degp[:, :, None], m2)
        aggp = _agg_kernel()(ei5, hp)
        xc, sv = _comb_call(aggp, hp, dinv, m2, b, p[:, None])
        sel, sf2 = _topk_call(sv, bt2, m2)
        x, m2 = xc, sel
    return _final_call(x, sf2, bt2, Wm, bm)


# speedup vs baseline: 49.8855x; 1.0320x over previous
import functools

import jax
import jax.numpy as jnp
from jax import lax
from jax.experimental import pallas as pl
from jax.experimental.pallas import tpu as pltpu
from jax.experimental.pallas import tpu_sc as plsc

N = 10000
E = 320000
F = 128
G = 16
NC, NS = 2, 16
NW = NC * NS
CH = 128
NCH = 80
EPT = NCH * CH
E2 = NW * EPT
NPAD = 10240
ROWS_PT = NPAD // NS
R = 1000
NBLK = N // R
IDX_NB = 8
AGG_NB = 4
F2 = F // 2
NCH2 = E2 // (NS * CH)
NROWS_T = N // NS

@functools.cache
def _sc_mesh():
    return plsc.VectorSubcoreMesh(core_axis_name="c", subcore_axis_name="s",
                                  num_cores=NC, num_subcores=NS)


def _deg_body(ei_hbm, m_hbm, out_hbm, m_v, sd_all, vals, zv, deg_sh, sem):
    c = lax.axis_index("c")
    s = lax.axis_index("s")
    w = s * NC + c
    pltpu.sync_copy(m_hbm, m_v)
    pltpu.sync_copy(ei_hbm.at[w], sd_all)
    for j in range(ROWS_PT // 16):
        zv[pl.ds(j * 16, 16)] = jnp.zeros((16,), jnp.float32)
    pltpu.sync_copy(zv, deg_sh.at[pl.ds(s * ROWS_PT, ROWS_PT)])
    plsc.subcore_barrier()

    def body(i, carry):
        for j in range(CH // 16):
            idx = sd_all[i, 0, pl.ds(j * 16, 16)]
            vals[pl.ds(j * 16, 16)] = plsc.load_gather(m_v, [idx])
        pltpu.sync_copy(vals, deg_sh.at[sd_all.at[i].at[1]], add=True)
        return carry

    lax.fori_loop(0, NCH, body, 0)
    plsc.subcore_barrier()
    pltpu.sync_copy(deg_sh.at[pl.ds(s * ROWS_PT, ROWS_PT)],
                    out_hbm.at[c, pl.ds(s * ROWS_PT, ROWS_PT)])


@functools.cache
def _deg_kernel():
    return pl.kernel(
        _deg_body,
        out_type=jax.ShapeDtypeStruct((NC, NPAD), jnp.float32),
        mesh=_sc_mesh(),
        compiler_params=pltpu.CompilerParams(use_tc_tiling_on_sc=False, needs_layout_passes=False),
        scratch_types=[
            pltpu.VMEM((N,), jnp.float32),
            pltpu.VMEM((NCH, 2, CH), jnp.int32),
            pltpu.VMEM((CH,), jnp.float32),
            pltpu.VMEM((ROWS_PT,), jnp.float32),
            pltpu.VMEM_SHARED((NPAD,), jnp.float32),
            pltpu.SemaphoreType.DMA,
        ],
    )


def _agg_body(ei_hbm, hp_hbm, out_hbm, sd, rows, zrow, hp_sh, agg_sh, semi,
              semg, sems):
    c = lax.axis_index("c")
    s = lax.axis_index("s")
    pltpu.sync_copy(hp_hbm.at[c, pl.ds(s * NROWS_T, NROWS_T), :],
                    hp_sh.at[pl.ds(s * NROWS_T, NROWS_T), :])
    for i in range(16):
        for j in range(F2 // 16):
            zrow[i, pl.ds(j * 16, 16)] = jnp.zeros((16,), jnp.float32)

    def zloop(i, carry):
        pltpu.sync_copy(zrow, agg_sh.at[pl.ds(s * ROWS_PT + i * 16, 16), :])
        return carry

    lax.fori_loop(0, ROWS_PT // 16, zloop, 0)
    plsc.subcore_barrier()

    for b in range(6):
        pltpu.async_copy(ei_hbm.at[s, b], sd.at[b], semi.at[b])
    for b in range(AGG_NB - 2):
        pltpu.make_async_copy(ei_hbm.at[s, b], sd.at[b], semi.at[b]).wait()
        pltpu.async_copy(hp_sh.at[sd.at[b].at[0]], rows.at[b], semg.at[b])

    def outer(j, carry):
        for b in range(IDX_NB):
            i = j * IDX_NB + b
            r = b % AGG_NB
            b2 = (b + 2) % IDX_NB
            r2 = (b + 2) % AGG_NB
            pltpu.make_async_copy(hp_sh.at[sd.at[b].at[0]], rows.at[r],
                                  semg.at[r]).wait()
            pltpu.async_copy(rows.at[r], agg_sh.at[sd.at[b].at[1]],
                             sems.at[r], add=True)

            @pl.when(i >= 2)
            def _():
                pltpu.make_async_copy(
                    rows.at[r2], agg_sh.at[sd.at[(b + 6) % IDX_NB].at[1]],
                    sems.at[r2]).wait()

            @pl.when(i + 2 < NCH2)
            def _():
                pltpu.make_async_copy(ei_hbm.at[s, i + 2], sd.at[b2],
                                      semi.at[b2]).wait()
                pltpu.async_copy(hp_sh.at[sd.at[b2].at[0]], rows.at[r2],
                                 semg.at[r2])

            @pl.when(i + 6 < NCH2)
            def _():
                pltpu.async_copy(ei_hbm.at[s, i + 6], sd.at[(b + 6) % IDX_NB],
                                 semi.at[(b + 6) % IDX_NB])
        return carry

    lax.fori_loop(0, NCH2 // IDX_NB, outer, 0)
    for b in range(2):
        bb = (NCH2 - 2 + b) % IDX_NB
        rr = (NCH2 - 2 + b) % AGG_NB
        pltpu.make_async_copy(rows.at[rr], agg_sh.at[sd.at[bb].at[1]],
                              sems.at[rr]).wait()
    plsc.subcore_barrier()
    pltpu.sync_copy(agg_sh.at[pl.ds(s * ROWS_PT, ROWS_PT), :],
                    out_hbm.at[c, pl.ds(s * ROWS_PT, ROWS_PT), :])


@functools.cache
def _agg_kernel():
    return pl.kernel(
        _agg_body,
        out_type=jax.ShapeDtypeStruct((NC, NPAD, F2), jnp.float32),
        mesh=_sc_mesh(),
        compiler_params=pltpu.CompilerParams(use_tc_tiling_on_sc=False, needs_layout_passes=False),
        scratch_types=[
            pltpu.VMEM((IDX_NB, 2, CH), jnp.int32),
            pltpu.VMEM((AGG_NB, CH, F2), jnp.float32),
            pltpu.VMEM((16, F2), jnp.float32),
            pltpu.VMEM_SHARED((N, F2), jnp.float32),
            pltpu.VMEM_SHARED((NPAD, F2), jnp.float32),
            pltpu.SemaphoreType.DMA((IDX_NB,)),
            pltpu.SemaphoreType.DMA((AGG_NB,)),
            pltpu.SemaphoreType.DMA((AGG_NB,)),
        ],
    )


def _hp_body(x_ref, w_ref, sf_ref, degp_ref, m_ref, hp_ref, dinv_ref):
    xb = x_ref[...] * sf_ref[...]
    h = jnp.dot(xb, w_ref[...], preferred_element_type=jnp.float32)
    deg = 1.0 + degp_ref[0] + degp_ref[1]
    dinv = lax.rsqrt(deg)
    hp = h * (dinv * m_ref[...])
    hp_ref[0] = hp[:, :F2]
    hp_ref[1] = hp[:, F2:]
    dinv_ref[...] = dinv


def _hp_call(x, w, sf, degp, m):
    return pl.pallas_call(
        _hp_body,
        grid=(NBLK,),
        in_specs=[
            pl.BlockSpec((R, F), lambda i: (i, 0)),
            pl.BlockSpec((F, F), lambda i: (0, 0)),
            pl.BlockSpec((R, 1), lambda i: (i, 0)),
            pl.BlockSpec((NC, R, 1), lambda i: (0, i, 0)),
            pl.BlockSpec((R, 1), lambda i: (i, 0)),
        ],
        out_specs=[
            pl.BlockSpec((NC, R, F2), lambda i: (0, i, 0)),
            pl.BlockSpec((R, 1), lambda i: (i, 0)),
        ],
        out_shape=[
            jax.ShapeDtypeStruct((NC, N, F2), jnp.float32),
            jax.ShapeDtypeStruct((N, 1), jnp.float32),
        ],
    )(x, w, sf, degp, m)


def _comb_body(aggp_ref, hp_ref, dinv_ref, m_ref, b_ref, p_ref, xc_ref,
               sv_ref):
    agg = jnp.concatenate([aggp_ref[0] + hp_ref[0],
                           aggp_ref[1] + hp_ref[1]], axis=1)
    xc = agg * dinv_ref[...] + b_ref[...]
    xc = jnp.maximum(xc, 0.0) * m_ref[...]
    xc_ref[...] = xc
    p = p_ref[...]
    pn = 1.0 / jnp.sqrt(jnp.sum(p * p))
    mv = lax.dot_general(xc, p, (((1,), (0,)), ((), ())),
                         preferred_element_type=jnp.float32)
    sv_ref[...] = jnp.tanh(mv * pn)


def _comb_call(aggp, hp, dinv, m, b, p):
    return pl.pallas_call(
        _comb_body,
        grid=(NBLK,),
        in_specs=[
            pl.BlockSpec((NC, R, F2), lambda i: (0, i, 0)),
            pl.BlockSpec((NC, R, F2), lambda i: (0, i, 0)),
            pl.BlockSpec((R, 1), lambda i: (i, 0)),
            pl.BlockSpec((R, 1), lambda i: (i, 0)),
            pl.BlockSpec((F,), lambda i: (0,)),
            pl.BlockSpec((F, 1), lambda i: (0, 0)),
        ],
        out_specs=[
            pl.BlockSpec((R, F), lambda i: (i, 0)),
            pl.BlockSpec((R, 1), lambda i: (i, 0)),
        ],
        out_shape=[
            jax.ShapeDtypeStruct((N, F), jnp.float32),
            jax.ShapeDtypeStruct((N, 1), jnp.float32),
        ],
    )(aggp, hp, dinv, m, b, p)


def _topk_body(sv_ref, batch_ref, m_ref, sel_ref, sfac_ref):
    f32, u32 = jnp.float32, jnp.uint32
    sv = sv_ref[...]
    valid = m_ref[...] > 0.0
    oh = batch_ref[...] == lax.broadcasted_iota(jnp.int32, (1, G), 1)
    validg = (valid & oh).astype(f32)
    dn = (((1,), (0,)), ((), ()))

    def nsum(x):
        return lax.dot_general(validg, x, (((0,), (0,)), ((), ())),
                               preferred_element_type=f32)

    def bcast(tg):
        return lax.dot_general(validg, tg, dn, preferred_element_type=f32)

    counts = nsum(jnp.ones((N, 1), f32))[:, 0]
    k = jnp.floor((counts + 1.0) * 0.5)

    ub = lax.bitcast_convert_type(sv, u32)
    neg = ub >= u32(0x80000000)
    ukey = jnp.where(neg, ~ub, ub | u32(0x80000000))
    ukey = jnp.where(valid, ukey, u32(0))

    jvec = lax.broadcasted_iota(jnp.int32, (1, 16), 1).astype(u32)

    def rnd(t, tv):
        shift = (u32(7) - t.astype(u32)) * u32(4)
        th = jnp.stack([(tv & u32(0xFFFF)).astype(f32),
                        (tv >> u32(16)).astype(f32)], axis=1)
        tn = bcast(th)
        tnode = (tn[:, 1:2].astype(u32) << u32(16)) | tn[:, 0:1].astype(u32)
        cand = tnode | (jvec << shift)
        cge = nsum((ukey >= cand).astype(f32))
        ge = (cge >= k[:, None]).astype(f32)
        bestj = (jnp.sum(ge, axis=1) - 1.0).astype(u32)
        return tv | (bestj << shift)

    tv = lax.fori_loop(0, 8, rnd, jnp.zeros((G,), u32))

    th = jnp.stack([(tv & u32(0xFFFF)).astype(f32),
                    (tv >> u32(16)).astype(f32)], axis=1)
    tn = bcast(th)
    tnode = (tn[:, 1:2].astype(u32) << u32(16)) | tn[:, 0:1].astype(u32)
    gttn = ((ukey > tnode) & valid).astype(f32)
    eqn = ((ukey == tnode) & valid).astype(f32)
    need = k - nsum(gttn)[:, 0]

    idx = lax.broadcasted_iota(jnp.int32, (N, 1), 0)
    jvec2 = lax.broadcasted_iota(jnp.int32, (1, 128), 1)

    def rnd2(t, iv):
        shift = 7 * (1 - t)
        ivn = bcast(iv[:, None]).astype(jnp.int32)
        cand2 = ivn + (jvec2 << shift)
        cle = lax.dot_general(validg * eqn, (idx < cand2).astype(f32),
                              (((0,), (0,)), ((), ())),
                              preferred_element_type=f32)
        le = (cle <= need[:, None]).astype(f32)
        bestj = jnp.sum(le, axis=1) - 1.0
        return iv + bestj * (jnp.float32(1) + jnp.float32(127) *
                             (1 - t).astype(f32))

    iv = lax.fori_loop(0, 2, rnd2, jnp.zeros((G,), f32))

    ivn = bcast(iv[:, None]).astype(jnp.int32)
    tie = eqn * (idx < ivn).astype(f32)
    sel = jnp.minimum(gttn + tie, 1.0)
    sel_ref[...] = sel
    sfac_ref[...] = sel * sv


def _topk_call(sv, batch, m):
    return pl.pallas_call(
        _topk_body,
        compiler_params=pltpu.CompilerParams(
            vmem_limit_bytes=100 * 1024 * 1024),
        out_shape=[
            jax.ShapeDtypeStruct((N, 1), jnp.float32),
            jax.ShapeDtypeStruct((N, 1), jnp.float32),
        ],
    )(sv, batch, m)


def _final_body(xc_ref, sfac_ref, batch_ref, wm_ref, bm_ref, out_ref):
    xs = xc_ref[...] * sfac_ref[...]
    oh = (batch_ref[...] ==
          lax.broadcasted_iota(jnp.int32, (1, G), 1)).astype(jnp.float32)
    pooled = lax.dot_general(oh, xs, (((0,), (0,)), ((), ())),
                             preferred_element_type=jnp.float32)
    out_ref[...] = jnp.dot(pooled, wm_ref[...],
                           preferred_element_type=jnp.float32) + bm_ref[...]


def _final_call(xc, sfac, batch, wm, bm):
    return pl.pallas_call(
        _final_body,
        out_shape=jax.ShapeDtypeStruct((G, wm.shape[1]), jnp.float32),
    )(xc, sfac, batch, wm, bm)


def kernel(x, edge_index, batch, W0, b0, p0, W1, b1, p1, W2, b2, p2, Wm, bm):
    srcp = jnp.concatenate([edge_index[0], jnp.zeros((E2 - E,), jnp.int32)])
    dstp = jnp.concatenate(
        [edge_index[1], jnp.full((E2 - E,), NPAD - 1, jnp.int32)])
    ei4 = jnp.stack([srcp, dstp]).reshape(2, NW, NCH, CH).transpose(1, 2, 0, 3)
    ei5 = jnp.stack([srcp, dstp]).reshape(2, NS, NCH2, CH).transpose(
        1, 2, 0, 3)
    bt2 = batch[:, None]
    m2 = jnp.ones((N, 1), jnp.float32)
    sf2 = jnp.ones((N, 1), jnp.float32)
    for (w, b, p) in ((W0, b0, p0), (W1, b1, p1), (W2, b2, p2)):
        degp = _deg_kernel()(ei4, m2.reshape(N))
        hp, dinv = _hp_call(x, w, sf2, degp[:, :, None], m2)
        aggp = _agg_kernel()(ei5, hp)
        xc, sv = _comb_call(aggp, hp, dinv, m2, b, p[:, None])
        sel, sf2 = _topk_call(sv, bt2, m2)
        x, m2 = xc, sel
    return _final_call(x, sf2, bt2, Wm, bm)
